# Initial kernel scaffold; baseline (speedup 1.0000x reference)
import functools
import jax
import jax.numpy as jnp
from jax import lax
from jax.experimental import pallas as pl
from jax.experimental.pallas import tpu as pltpu
from jax.experimental.pallas import tpu_sc as plsc

IN_F = 16384
OUT_F = 4096
NNZ = 131072
B = 256
EPS = 1e-5
NEG_SLOPE = 0.01

NC = 2
NS = 16
L = 16

C = 128
EPT = NNZ // (NC * NS)
NCHUNK = EPT // C
ROWS_PER_TILE = OUT_F // NS


def _sc_body(ft_hbm, inidx_hbm, outidx_hbm, w_hbm, part_hbm,
             idx_v, oidx_v, w_v, g_v, acc_sh, sem):
    c = lax.axis_index("c")
    s = lax.axis_index("s")
    wid = s * NC + c

    def zrow(i, carry):
        z = jnp.zeros((L,), jnp.float32)
        for j in range(B // L):
            g_v[i, pl.ds(j * L, L)] = z
        return carry

    lax.fori_loop(0, C, zrow, 0)
    for r in range(ROWS_PER_TILE // C):
        pltpu.sync_copy(g_v, acc_sh.at[pl.ds(s * ROWS_PER_TILE + r * C, C)])
    plsc.subcore_barrier()

    def chunk(k, carry):
        base = wid * EPT + k * C
        pltpu.sync_copy(inidx_hbm.at[pl.ds(base, C)], idx_v)
        pltpu.sync_copy(outidx_hbm.at[pl.ds(base, C)], oidx_v)
        pltpu.sync_copy(w_hbm.at[pl.ds(base, C)], w_v)
        pltpu.async_copy(ft_hbm.at[idx_v], g_v, sem).wait()

        def mrow(e, cc):
            we = w_v[e]
            for j in range(B // L):
                sl = pl.ds(j * L, L)
                g_v[e, sl] = g_v[e, sl] * we
            return cc

        lax.fori_loop(0, C, mrow, 0)
        pltpu.sync_copy(g_v, acc_sh.at[oidx_v], add=True)
        return carry

    lax.fori_loop(0, NCHUNK, chunk, 0)
    plsc.subcore_barrier()

    pltpu.sync_copy(acc_sh.at[pl.ds(s * ROWS_PER_TILE, ROWS_PER_TILE)],
                    part_hbm.at[c, pl.ds(s * ROWS_PER_TILE, ROWS_PER_TILE)])


def _sc_call(ft, in_idx, out_idx, weights):
    mesh = plsc.VectorSubcoreMesh(core_axis_name="c", subcore_axis_name="s",
                                  num_cores=NC, num_subcores=NS)
    f = pl.kernel(
        _sc_body,
        out_type=jax.ShapeDtypeStruct((NC, OUT_F, B), jnp.float32),
        mesh=mesh,
        scratch_types=[
            pltpu.VMEM((C,), jnp.int32),
            pltpu.VMEM((C,), jnp.int32),
            pltpu.VMEM((C,), jnp.float32),
            pltpu.VMEM((C, B), jnp.float32),
            pltpu.VMEM_SHARED((OUT_F, B), jnp.float32),
            pltpu.SemaphoreType.DMA,
        ],
    )
    return f(ft, in_idx, out_idx, weights)


def _tc_body(p_ref, bias_ref, o_ref):
    y = p_ref[0] + p_ref[1] + bias_ref[0][:, None]
    mean = jnp.mean(y, axis=1, keepdims=True)
    d = y - mean
    var = jnp.mean(d * d, axis=1, keepdims=True)
    yn = d * lax.rsqrt(var + EPS)
    yn = jnp.where(yn >= 0, yn, NEG_SLOPE * yn)
    o_ref[...] = yn.T


def _tc_call(parts, bias):
    bo = 256
    grid = OUT_F // bo
    return pl.pallas_call(
        _tc_body,
        grid=(grid,),
        in_specs=[
            pl.BlockSpec((NC, bo, B), lambda i: (0, i, 0)),
            pl.BlockSpec((1, bo), lambda i: (0, i)),
        ],
        out_specs=pl.BlockSpec((B, bo), lambda i: (0, i)),
        out_shape=jax.ShapeDtypeStruct((B, OUT_F), jnp.float32),
    )(parts, bias)


@jax.jit
def kernel(features, in_idx, out_idx, weights, bias):
    ft = features.T
    parts = _sc_call(ft, in_idx.astype(jnp.int32), out_idx.astype(jnp.int32),
                     weights)
    return _tc_call(parts, bias.reshape(1, OUT_F))


# speedup vs baseline: 1.5066x; 1.5066x over previous
import jax
import jax.numpy as jnp
from jax import lax
from jax.experimental import pallas as pl
from jax.experimental.pallas import tpu as pltpu
from jax.experimental.pallas import tpu_sc as plsc

IN_F = 16384
OUT_F = 4096
NNZ = 131072
B = 256
EPS = 1e-5
NEG_SLOPE = 0.01

NC = 2
NS = 16
NW = NC * NS
L = 16

SB = 4
NPASS = B // (SB * NW)
E = 2048
NCHUNK = NNZ // E
NGRP = E // L


def _sc_body(feat_hbm, inidx_hbm, outidx_hbm, w_hbm, y_hbm,
             fs_v, acc_v, idx_v, oidx_v, w_v, sem):
    c = lax.axis_index("c")
    s = lax.axis_index("s")
    wid = s * NC + c

    for p in range(NPASS):
        b0 = wid * SB + p * (SB * NW)
        pltpu.async_copy(feat_hbm.at[pl.ds(b0 * IN_F, SB * IN_F)], fs_v,
                         sem).wait()
        z = jnp.zeros((L,), jnp.float32)

        def zcol(j, cc):
            acc_v[pl.ds(j * L, L)] = z
            return cc

        lax.fori_loop(0, SB * OUT_F // L, zcol, 0)

        def chunk(k, cc):
            base = k * E
            pltpu.sync_copy(inidx_hbm.at[pl.ds(base, E)], idx_v)
            pltpu.sync_copy(outidx_hbm.at[pl.ds(base, E)], oidx_v)
            pltpu.sync_copy(w_hbm.at[pl.ds(base, E)], w_v)

            def grp(g, gc):
                sl = pl.ds(g * L, L)
                ii = idx_v[sl]
                oo = oidx_v[sl]
                ww = w_v[sl]
                for b in range(SB):
                    vals = plsc.load_gather(fs_v, [ii + (b * IN_F)]) * ww
                    plsc.addupdate_scatter(acc_v, [oo + (b * OUT_F)], vals)
                return gc

            lax.fori_loop(0, NGRP, grp, 0)
            return cc

        lax.fori_loop(0, NCHUNK, chunk, 0)

        pltpu.sync_copy(acc_v, y_hbm.at[pl.ds(b0 * OUT_F, SB * OUT_F)])


def _sc_call(features, in_idx, out_idx, weights):
    mesh = plsc.VectorSubcoreMesh(core_axis_name="c", subcore_axis_name="s",
                                  num_cores=NC, num_subcores=NS)
    f = pl.kernel(
        _sc_body,
        out_type=jax.ShapeDtypeStruct((B * OUT_F,), jnp.float32),
        mesh=mesh,
        compiler_params=pltpu.CompilerParams(needs_layout_passes=False),
        scratch_types=[
            pltpu.VMEM((SB * IN_F,), jnp.float32),
            pltpu.VMEM((SB * OUT_F,), jnp.float32),
            pltpu.VMEM((E,), jnp.int32),
            pltpu.VMEM((E,), jnp.int32),
            pltpu.VMEM((E,), jnp.float32),
            pltpu.SemaphoreType.DMA,
        ],
    )
    return f(features.reshape(B * IN_F), in_idx, out_idx,
             weights).reshape(B, OUT_F)


def _tc_body(y_ref, bias_ref, o_ref):
    y = y_ref[...] + bias_ref[...]
    mean = jnp.mean(y, axis=0, keepdims=True)
    d = y - mean
    var = jnp.mean(d * d, axis=0, keepdims=True)
    yn = d * lax.rsqrt(var + EPS)
    o_ref[...] = jnp.where(yn >= 0, yn, NEG_SLOPE * yn)


def _tc_call(y_lin, bias):
    bo = 512
    return pl.pallas_call(
        _tc_body,
        grid=(OUT_F // bo,),
        in_specs=[
            pl.BlockSpec((B, bo), lambda i: (0, i)),
            pl.BlockSpec((1, bo), lambda i: (0, i)),
        ],
        out_specs=pl.BlockSpec((B, bo), lambda i: (0, i)),
        out_shape=jax.ShapeDtypeStruct((B, OUT_F), jnp.float32),
    )(y_lin, bias)


@jax.jit
def kernel(features, in_idx, out_idx, weights, bias):
    y_lin = _sc_call(features, in_idx.astype(jnp.int32),
                     out_idx.astype(jnp.int32), weights)
    return _tc_call(y_lin, bias.reshape(1, OUT_F))


# speedup vs baseline: 5.3894x; 3.5771x over previous
import jax
import jax.numpy as jnp
from jax import lax
from jax.experimental import pallas as pl
from jax.experimental.pallas import tpu as pltpu
from jax.experimental.pallas import tpu_sc as plsc

IN_F = 16384
OUT_F = 4096
NNZ = 131072
B = 256
EPS = 1e-5
NEG_SLOPE = 0.01

NC = 2
NS = 16
NW = NC * NS
L = 16

SB = 4
NPASS = B // (SB * NW)
E = 4096
NCHUNK = NNZ // E
NGRP = E // L


def _sc_body(feat_hbm, packed_hbm, y_hbm, fs_v, acc_v, eb0, eb1, sem, fsem):
    c = lax.axis_index("c")
    s = lax.axis_index("s")
    wid = s * NC + c

    def start(k, buf):
        pltpu.async_copy(packed_hbm.at[pl.ds(k * 3 * E, 3 * E)], buf, sem)

    def wait(buf):
        pltpu.make_async_copy(packed_hbm.at[pl.ds(0, 3 * E)], buf, sem).wait()

    def compute(eb):
        @plsc.parallel_loop(0, NGRP, step=1, unroll=4)
        def grp(g):
            ii = eb[pl.ds(g * L, L)]
            oo = eb[pl.ds(E + g * L, L)]
            ww = plsc.bitcast(eb[pl.ds(2 * E + g * L, L)], jnp.float32)
            for b in range(SB):
                vals = plsc.load_gather(fs_v, [ii + (b * IN_F)]) * ww
                plsc.addupdate_scatter(acc_v, [oo + (b * OUT_F)], vals)

    for p in range(NPASS):
        b0 = wid * SB + p * (SB * NW)
        fdma = pltpu.async_copy(feat_hbm.at[pl.ds(b0 * IN_F, SB * IN_F)],
                                fs_v, fsem)
        z = jnp.zeros((L,), jnp.float32)

        def zcol(j, cc):
            for u in range(8):
                acc_v[pl.ds((j * 8 + u) * L, L)] = z
            return cc

        lax.fori_loop(0, SB * OUT_F // (8 * L), zcol, 0)
        fdma.wait()

        start(0, eb0)
        start(1, eb1)

        def outer(t, cc):
            wait(eb0)
            compute(eb0)
            start(2 * t + 2, eb0)
            wait(eb1)
            compute(eb1)
            start(2 * t + 3, eb1)
            return cc

        lax.fori_loop(0, NCHUNK // 2 - 1, outer, 0)
        wait(eb0)
        compute(eb0)
        wait(eb1)
        compute(eb1)

        pltpu.sync_copy(acc_v, y_hbm.at[pl.ds(b0 * OUT_F, SB * OUT_F)])


def _sc_call(features, packed):
    mesh = plsc.VectorSubcoreMesh(core_axis_name="c", subcore_axis_name="s",
                                  num_cores=NC, num_subcores=NS)
    f = pl.kernel(
        _sc_body,
        out_type=jax.ShapeDtypeStruct((B * OUT_F,), jnp.float32),
        mesh=mesh,
        compiler_params=pltpu.CompilerParams(needs_layout_passes=False),
        scratch_types=[
            pltpu.VMEM((SB * IN_F,), jnp.float32),
            pltpu.VMEM((SB * OUT_F,), jnp.float32),
            pltpu.VMEM((3 * E,), jnp.int32),
            pltpu.VMEM((3 * E,), jnp.int32),
            pltpu.SemaphoreType.DMA,
            pltpu.SemaphoreType.DMA,
        ],
    )
    return f(features.reshape(B * IN_F), packed).reshape(B, OUT_F)


def _tc_body(y_ref, bias_ref, o_ref):
    y = y_ref[...] + bias_ref[...]
    mean = jnp.mean(y, axis=0, keepdims=True)
    d = y - mean
    var = jnp.mean(d * d, axis=0, keepdims=True)
    yn = d * lax.rsqrt(var + EPS)
    o_ref[...] = jnp.where(yn >= 0, yn, NEG_SLOPE * yn)


def _tc_call(y_lin, bias):
    bo = 512
    return pl.pallas_call(
        _tc_body,
        grid=(OUT_F // bo,),
        in_specs=[
            pl.BlockSpec((B, bo), lambda i: (0, i)),
            pl.BlockSpec((1, bo), lambda i: (0, i)),
        ],
        out_specs=pl.BlockSpec((B, bo), lambda i: (0, i)),
        out_shape=jax.ShapeDtypeStruct((B, OUT_F), jnp.float32),
    )(y_lin, bias)


@jax.jit
def kernel(features, in_idx, out_idx, weights, bias):
    in_idx = in_idx.astype(jnp.int32)
    out_idx = out_idx.astype(jnp.int32)
    w_bits = lax.bitcast_convert_type(weights, jnp.int32)
    packed = jnp.stack([in_idx.reshape(NCHUNK, E),
                        out_idx.reshape(NCHUNK, E),
                        w_bits.reshape(NCHUNK, E)], axis=1).reshape(-1)
    y_lin = _sc_call(features, packed)
    return _tc_call(y_lin, bias.reshape(1, OUT_F))
